# Initial kernel scaffold; baseline (speedup 1.0000x reference)
import jax
import jax.numpy as jnp
from jax.experimental import pallas as pl


def kernel(x, W_enc, b_enc, tao, n, k, q):
    raise NotImplementedError("write your pallas kernel here")


# speedup vs baseline: 3.1630x; 3.1630x over previous
import jax
import jax.numpy as jnp
import numpy as np
from jax.experimental import pallas as pl

K_NEIGHBORS = 10
N, KSHOT, Q = 100, 5, 15
D_IN, D_OUT = 2048, 1024
NQ = N * Q
NQP = 1536
NP = 128
NSUP = N * KSHOT
NSUPP = 512
ROWS = N * (KSHOT + Q)
BLK = 128
NEG = -1e30
BIG = 1e30


def _enc_kernel(x_ref, w_ref, b_ref, out_ref):
    out_ref[:] = (
        jnp.dot(x_ref[:], w_ref[:], preferred_element_type=jnp.float32) + b_ref[:]
    )


def _proto_kernel(sup_ref, q_ref, qn_out, apn_out):
    sup = sup_ref[:]
    qf = q_ref[:]

    proto = (
        sup[0 * NP:1 * NP] + sup[1 * NP:2 * NP] + sup[2 * NP:3 * NP]
        + sup[3 * NP:4 * NP] + sup[4 * NP:5 * NP]
    ) / float(KSHOT)
    pnorm = jnp.sqrt(jnp.sum(proto * proto, axis=1, keepdims=True))
    pn = proto / (pnorm + 1e-8)
    qnorm = jnp.sqrt(jnp.sum(qf * qf, axis=1, keepdims=True))
    qn = qf / (qnorm + 1e-8)
    qn_out[:] = qn

    pre = jax.lax.dot_general(
        qn, pn, (((1,), (1,)), ((), ())), preferred_element_type=jnp.float32
    )
    colid = jax.lax.broadcasted_iota(jnp.int32, (NQP, NP), 1)
    rowid = jax.lax.broadcasted_iota(jnp.int32, (NQP, NP), 0)
    pre_m = jnp.where(colid < N, pre, NEG)
    rowmax = jnp.max(pre_m, axis=1, keepdims=True)
    idx = jnp.where(pre_m == rowmax, colid, jnp.int32(2**30))
    amin = jnp.min(idx, axis=1, keepdims=True)
    onehot = (colid == amin) & (rowid < NQ)
    exp_ref_w = jnp.where(onehot, jnp.exp(pre), 0.0)

    self_sim = jnp.sum(pn * pn, axis=1, keepdims=True)
    exp_self = jnp.exp(self_sim)
    ones = jnp.ones((NQP, 1), dtype=jnp.float32)
    denom = jax.lax.dot_general(
        exp_ref_w, ones, (((0,), (0,)), ((), ())), preferred_element_type=jnp.float32
    ) + exp_self
    num = jax.lax.dot_general(
        exp_ref_w, qf, (((0,), (0,)), ((), ())), preferred_element_type=jnp.float32
    ) + exp_self * proto
    ap = num / denom
    apnorm = jnp.sqrt(jnp.sum(ap * ap, axis=1, keepdims=True))
    apn_out[:] = ap / (apnorm + 1e-8)


def _sim_kernel(qn_blk_ref, qn_all_ref, sim_out, thr_out):
    i = pl.program_id(0)
    a = qn_blk_ref[:]
    b = qn_all_ref[:]
    sim = jax.lax.dot_general(
        a, b, (((1,), (1,)), ((), ())), preferred_element_type=jnp.float32
    )
    sim_out[:] = sim
    colid = jax.lax.broadcasted_iota(jnp.int32, (BLK, NQP), 1)
    work = jnp.where(colid < NQ, sim, NEG)
    thr = jnp.full((BLK, 1), NEG, dtype=jnp.float32)
    for _ in range(K_NEIGHBORS):
        thr = jnp.max(work, axis=1, keepdims=True)
        work = jnp.where(work >= thr, NEG, work)
    rowg = i * BLK + jax.lax.broadcasted_iota(jnp.int32, (BLK, 1), 0)
    thr_out[:] = jnp.where(rowg < NQ, thr, BIG)


def _agg_kernel(sim_ref, thr_i_ref, thr_j_ref, q_ref, apn_ref, tao_ref, out_ref):
    sim = sim_ref[:]
    ti = thr_i_ref[:]
    tj = thr_j_ref[:]
    w = jnp.where((sim >= ti) & (sim >= tj), jnp.exp(sim), 0.0)
    s = jnp.sum(w, axis=1, keepdims=True)
    s = jnp.where(s > 0.0, s, 1.0)
    aq = jnp.dot(w, q_ref[:], preferred_element_type=jnp.float32) / s
    anorm = jnp.sqrt(jnp.sum(aq * aq, axis=1, keepdims=True))
    aqn = aq / (anorm + 1e-8)
    out_ref[:] = tao_ref[0, 0] * jax.lax.dot_general(
        aqn, apn_ref[:], (((1,), (1,)), ((), ())), preferred_element_type=jnp.float32
    )


def kernel(x, W_enc, b_enc, tao, n, k, q):
    f32 = jnp.float32

    feat = pl.pallas_call(
        _enc_kernel,
        grid=(8,),
        in_specs=[
            pl.BlockSpec((256, D_IN), lambda i: (i, 0)),
            pl.BlockSpec((D_IN, D_OUT), lambda i: (0, 0)),
            pl.BlockSpec((1, D_OUT), lambda i: (0, 0)),
        ],
        out_specs=pl.BlockSpec((256, D_OUT), lambda i: (i, 0)),
        out_shape=jax.ShapeDtypeStruct((ROWS, D_OUT), f32),
    )(x, W_enc, b_enc.reshape(1, D_OUT))

    f3 = feat.reshape(N, KSHOT + Q, D_OUT)
    sup3 = jnp.pad(f3[:, :KSHOT], ((0, NP - N), (0, 0), (0, 0)))
    sup = jnp.transpose(sup3, (1, 0, 2)).reshape(KSHOT * NP, D_OUT)
    qf = f3[:, KSHOT:].reshape(NQ, D_OUT)
    qf = jnp.pad(qf, ((0, NQP - NQ), (0, 0)))

    qn, apn = pl.pallas_call(
        _proto_kernel,
        out_shape=[
            jax.ShapeDtypeStruct((NQP, D_OUT), f32),
            jax.ShapeDtypeStruct((NP, D_OUT), f32),
        ],
    )(sup, qf)

    sim, thr = pl.pallas_call(
        _sim_kernel,
        grid=(NQP // BLK,),
        in_specs=[
            pl.BlockSpec((BLK, D_OUT), lambda i: (i, 0)),
            pl.BlockSpec((NQP, D_OUT), lambda i: (0, 0)),
        ],
        out_specs=[
            pl.BlockSpec((BLK, NQP), lambda i: (i, 0)),
            pl.BlockSpec((BLK, 1), lambda i: (i, 0)),
        ],
        out_shape=[
            jax.ShapeDtypeStruct((NQP, NQP), f32),
            jax.ShapeDtypeStruct((NQP, 1), f32),
        ],
    )(qn, qn)

    thr_row = thr.reshape(1, NQP)

    out = pl.pallas_call(
        _agg_kernel,
        grid=(NQP // BLK,),
        in_specs=[
            pl.BlockSpec((BLK, NQP), lambda i: (i, 0)),
            pl.BlockSpec((BLK, 1), lambda i: (i, 0)),
            pl.BlockSpec((1, NQP), lambda i: (0, 0)),
            pl.BlockSpec((NQP, D_OUT), lambda i: (0, 0)),
            pl.BlockSpec((NP, D_OUT), lambda i: (0, 0)),
            pl.BlockSpec((1, 1), lambda i: (0, 0)),
        ],
        out_specs=pl.BlockSpec((BLK, NP), lambda i: (i, 0)),
        out_shape=jax.ShapeDtypeStruct((NQP, NP), f32),
    )(sim, thr, thr_row, qf, apn, tao.reshape(1, 1))

    return out[:NQ, :N]


# speedup vs baseline: 3.3306x; 1.0530x over previous
import jax
import jax.numpy as jnp
from jax.experimental import pallas as pl
from jax.experimental.pallas import tpu as pltpu

K_NEIGHBORS = 10
N, KSHOT, Q = 100, 5, 15
D_IN, D_OUT = 2048, 1024
NQ = N * Q
NQP = 1536
NP = 128
NSUP = N * KSHOT
NSUPP = 512
ROWS = N * (KSHOT + Q)
BLK = 128
NEG = -1e30
BIG = 1e30


def _enc_kernel(x_ref, w_ref, b_ref, out_ref):
    out_ref[:] = (
        jnp.dot(x_ref[:], w_ref[:], preferred_element_type=jnp.float32) + b_ref[:]
    )


def _proto_kernel(sup_ref, q_ref, qn_out, apn_out):
    sup = sup_ref[:]
    qf = q_ref[:]

    proto = (
        sup[0 * NP:1 * NP] + sup[1 * NP:2 * NP] + sup[2 * NP:3 * NP]
        + sup[3 * NP:4 * NP] + sup[4 * NP:5 * NP]
    ) / float(KSHOT)
    pnorm = jnp.sqrt(jnp.sum(proto * proto, axis=1, keepdims=True))
    pn = proto / (pnorm + 1e-8)
    qnorm = jnp.sqrt(jnp.sum(qf * qf, axis=1, keepdims=True))
    qn = qf / (qnorm + 1e-8)
    qn_out[:] = qn

    pre = jax.lax.dot_general(
        qn, pn, (((1,), (1,)), ((), ())), preferred_element_type=jnp.float32
    )
    colid = jax.lax.broadcasted_iota(jnp.int32, (NQP, NP), 1)
    rowid = jax.lax.broadcasted_iota(jnp.int32, (NQP, NP), 0)
    pre_m = jnp.where(colid < N, pre, NEG)
    rowmax = jnp.max(pre_m, axis=1, keepdims=True)
    idx = jnp.where(pre_m == rowmax, colid, jnp.int32(2**30))
    amin = jnp.min(idx, axis=1, keepdims=True)
    onehot = (colid == amin) & (rowid < NQ)
    exp_ref_w = jnp.where(onehot, jnp.exp(pre), 0.0)

    self_sim = jnp.sum(pn * pn, axis=1, keepdims=True)
    exp_self = jnp.exp(self_sim)
    ones = jnp.ones((NQP, 1), dtype=jnp.float32)
    denom = jax.lax.dot_general(
        exp_ref_w, ones, (((0,), (0,)), ((), ())), preferred_element_type=jnp.float32
    ) + exp_self
    num = jax.lax.dot_general(
        exp_ref_w, qf, (((0,), (0,)), ((), ())), preferred_element_type=jnp.float32
    ) + exp_self * proto
    ap = num / denom
    apnorm = jnp.sqrt(jnp.sum(ap * ap, axis=1, keepdims=True))
    apn_out[:] = ap / (apnorm + 1e-8)


NB = NQP // BLK


def _knn_kernel(qn_blk_ref, qn_all_ref, q_ref, apn_ref, tao_ref, out_ref,
                sim_scr, thrc_scr, thrr_scr):
    i = pl.program_id(0)

    @pl.when(i < NB)
    def _phase_sim():
        a = qn_blk_ref[:]
        b = qn_all_ref[:]
        sim = jax.lax.dot_general(
            a, b, (((1,), (1,)), ((), ())), preferred_element_type=jnp.float32
        )
        sim_scr[pl.ds(i * BLK, BLK), :] = sim
        colid = jax.lax.broadcasted_iota(jnp.int32, (BLK, NQP), 1)
        work = jnp.where(colid < NQ, sim, NEG)
        thr = jnp.full((BLK, 1), NEG, dtype=jnp.float32)
        for _ in range(K_NEIGHBORS):
            thr = jnp.max(work, axis=1, keepdims=True)
            work = jnp.where(work >= thr, NEG, work)
        rowg = i * BLK + jax.lax.broadcasted_iota(jnp.int32, (BLK, 1), 0)
        thr = jnp.where(rowg < NQ, thr, BIG)
        thrc_scr[pl.ds(i * BLK, BLK), :] = thr
        r = jax.lax.broadcasted_iota(jnp.int32, (BLK, BLK), 0)
        c = jax.lax.broadcasted_iota(jnp.int32, (BLK, BLK), 1)
        eye = (r == c).astype(jnp.float32)
        thr_row = jax.lax.dot_general(
            thr, eye, (((0,), (0,)), ((), ())), preferred_element_type=jnp.float32
        )
        thrr_scr[:, pl.ds(i * BLK, BLK)] = thr_row

    @pl.when(i >= NB)
    def _phase_agg():
        j = i - NB
        sim = sim_scr[pl.ds(j * BLK, BLK), :]
        ti = thrc_scr[pl.ds(j * BLK, BLK), :]
        tj = thrr_scr[:]
        w = jnp.where((sim >= ti) & (sim >= tj), jnp.exp(sim), 0.0)
        s = jnp.sum(w, axis=1, keepdims=True)
        s = jnp.where(s > 0.0, s, 1.0)
        aq = jnp.dot(w, q_ref[:], preferred_element_type=jnp.float32) / s
        anorm = jnp.sqrt(jnp.sum(aq * aq, axis=1, keepdims=True))
        aqn = aq / (anorm + 1e-8)
        out_ref[:] = tao_ref[0, 0] * jax.lax.dot_general(
            aqn, apn_ref[:], (((1,), (1,)), ((), ())),
            preferred_element_type=jnp.float32,
        )


def kernel(x, W_enc, b_enc, tao, n, k, q):
    f32 = jnp.float32

    feat = pl.pallas_call(
        _enc_kernel,
        grid=(8,),
        in_specs=[
            pl.BlockSpec((256, D_IN), lambda i: (i, 0)),
            pl.BlockSpec((D_IN, D_OUT), lambda i: (0, 0)),
            pl.BlockSpec((1, D_OUT), lambda i: (0, 0)),
        ],
        out_specs=pl.BlockSpec((256, D_OUT), lambda i: (i, 0)),
        out_shape=jax.ShapeDtypeStruct((ROWS, D_OUT), f32),
    )(x, W_enc, b_enc.reshape(1, D_OUT))

    f3 = feat.reshape(N, KSHOT + Q, D_OUT)
    sup3 = jnp.pad(f3[:, :KSHOT], ((0, NP - N), (0, 0), (0, 0)))
    sup = jnp.transpose(sup3, (1, 0, 2)).reshape(KSHOT * NP, D_OUT)
    qf = f3[:, KSHOT:].reshape(NQ, D_OUT)
    qf = jnp.pad(qf, ((0, NQP - NQ), (0, 0)))

    qn, apn = pl.pallas_call(
        _proto_kernel,
        out_shape=[
            jax.ShapeDtypeStruct((NQP, D_OUT), f32),
            jax.ShapeDtypeStruct((NP, D_OUT), f32),
        ],
    )(sup, qf)

    out = pl.pallas_call(
        _knn_kernel,
        grid=(2 * NB,),
        in_specs=[
            pl.BlockSpec((BLK, D_OUT), lambda i: (jnp.where(i < NB, i, NB - 1), 0)),
            pl.BlockSpec((NQP, D_OUT), lambda i: (0, 0)),
            pl.BlockSpec((NQP, D_OUT), lambda i: (0, 0)),
            pl.BlockSpec((NP, D_OUT), lambda i: (0, 0)),
            pl.BlockSpec((1, 1), lambda i: (0, 0)),
        ],
        out_specs=pl.BlockSpec((BLK, NP), lambda i: (jnp.where(i < NB, 0, i - NB), 0)),
        out_shape=jax.ShapeDtypeStruct((NQP, NP), f32),
        scratch_shapes=[
            pltpu.VMEM((NQP, NQP), f32),
            pltpu.VMEM((NQP, 1), f32),
            pltpu.VMEM((1, NQP), f32),
        ],
    )(qn, qn, qf, apn, tao.reshape(1, 1))

    return out[:NQ, :N]


# speedup vs baseline: 3.3819x; 1.0154x over previous
import jax
import jax.numpy as jnp
from jax.experimental import pallas as pl
from jax.experimental.pallas import tpu as pltpu

K_NEIGHBORS = 10
N, KSHOT, Q = 100, 5, 15
D_IN, D_OUT = 2048, 1024
NQ = N * Q
NQP = 1536
NP = 128
NSUP = N * KSHOT
NSUPP = 512
ROWS = N * (KSHOT + Q)
BLK = 128
NEG = -1e30
BIG = 1e30


def _enc_kernel(x_ref, w_ref, b_ref, out_ref):
    out_ref[:] = (
        jnp.dot(x_ref[:], w_ref[:], preferred_element_type=jnp.float32) + b_ref[:]
    )


def _proto_kernel(sup_ref, q_ref, qn_out, apn_out):
    sup = sup_ref[:]
    qf = q_ref[:]

    proto = (
        sup[0 * NP:1 * NP] + sup[1 * NP:2 * NP] + sup[2 * NP:3 * NP]
        + sup[3 * NP:4 * NP] + sup[4 * NP:5 * NP]
    ) / float(KSHOT)
    pnorm = jnp.sqrt(jnp.sum(proto * proto, axis=1, keepdims=True))
    pn = proto / (pnorm + 1e-8)
    qnorm = jnp.sqrt(jnp.sum(qf * qf, axis=1, keepdims=True))
    qn = qf / (qnorm + 1e-8)
    qn_out[:] = qn

    pre = jax.lax.dot_general(
        qn, pn, (((1,), (1,)), ((), ())), preferred_element_type=jnp.float32
    )
    colid = jax.lax.broadcasted_iota(jnp.int32, (NQP, NP), 1)
    rowid = jax.lax.broadcasted_iota(jnp.int32, (NQP, NP), 0)
    pre_m = jnp.where(colid < N, pre, NEG)
    rowmax = jnp.max(pre_m, axis=1, keepdims=True)
    idx = jnp.where(pre_m == rowmax, colid, jnp.int32(2**30))
    amin = jnp.min(idx, axis=1, keepdims=True)
    onehot = (colid == amin) & (rowid < NQ)
    exp_ref_w = jnp.where(onehot, jnp.exp(pre), 0.0)

    self_sim = jnp.sum(pn * pn, axis=1, keepdims=True)
    exp_self = jnp.exp(self_sim)
    ones = jnp.ones((NQP, 1), dtype=jnp.float32)
    denom = jax.lax.dot_general(
        exp_ref_w, ones, (((0,), (0,)), ((), ())), preferred_element_type=jnp.float32
    ) + exp_self
    num = jax.lax.dot_general(
        exp_ref_w, qf, (((0,), (0,)), ((), ())), preferred_element_type=jnp.float32
    ) + exp_self * proto
    ap = num / denom
    apnorm = jnp.sqrt(jnp.sum(ap * ap, axis=1, keepdims=True))
    apn_out[:] = ap / (apnorm + 1e-8)


NB = NQP // BLK


def _knn_kernel(qn_blk_ref, qn_all_ref, q_ref, apn_ref, tao_ref, out_ref,
                sim_scr, thrc_scr, thrr_scr):
    i = pl.program_id(0)

    @pl.when(i < NB)
    def _phase_sim():
        a = qn_blk_ref[:]
        b = qn_all_ref[:]
        sim = jax.lax.dot_general(
            a, b, (((1,), (1,)), ((), ())), preferred_element_type=jnp.float32
        )
        sim_scr[pl.ds(i * BLK, BLK), :] = sim
        colid = jax.lax.broadcasted_iota(jnp.int32, (BLK, NQP), 1)
        work = jnp.where(colid < NQ, sim, NEG)
        thr = jnp.full((BLK, 1), NEG, dtype=jnp.float32)
        for _ in range(K_NEIGHBORS):
            thr = jnp.max(work, axis=1, keepdims=True)
            work = jnp.where(work >= thr, NEG, work)
        rowg = i * BLK + jax.lax.broadcasted_iota(jnp.int32, (BLK, 1), 0)
        thr = jnp.where(rowg < NQ, thr, BIG)
        thrc_scr[pl.ds(i * BLK, BLK), :] = thr
        thrr_scr[:, pl.ds(i * BLK, BLK)] = jnp.transpose(thr, (1, 0))

    @pl.when(i >= NB)
    def _phase_agg():
        j = i - NB
        sim = sim_scr[pl.ds(j * BLK, BLK), :]
        ti = thrc_scr[pl.ds(j * BLK, BLK), :]
        tj = thrr_scr[:]
        w = jnp.where((sim >= ti) & (sim >= tj), jnp.exp(sim), 0.0)
        s = jnp.sum(w, axis=1, keepdims=True)
        s = jnp.where(s > 0.0, s, 1.0)
        aq = jnp.dot(w, q_ref[:], preferred_element_type=jnp.float32) / s
        anorm = jnp.sqrt(jnp.sum(aq * aq, axis=1, keepdims=True))
        aqn = aq / (anorm + 1e-8)
        out_ref[:] = tao_ref[0, 0] * jax.lax.dot_general(
            aqn, apn_ref[:], (((1,), (1,)), ((), ())),
            preferred_element_type=jnp.float32,
        )


def kernel(x, W_enc, b_enc, tao, n, k, q):
    f32 = jnp.float32

    feat = pl.pallas_call(
        _enc_kernel,
        grid=(8,),
        in_specs=[
            pl.BlockSpec((256, D_IN), lambda i: (i, 0)),
            pl.BlockSpec((D_IN, D_OUT), lambda i: (0, 0)),
            pl.BlockSpec((1, D_OUT), lambda i: (0, 0)),
        ],
        out_specs=pl.BlockSpec((256, D_OUT), lambda i: (i, 0)),
        out_shape=jax.ShapeDtypeStruct((ROWS, D_OUT), f32),
    )(x, W_enc, b_enc.reshape(1, D_OUT))

    f3 = feat.reshape(N, KSHOT + Q, D_OUT)
    sup3 = jnp.pad(f3[:, :KSHOT], ((0, NP - N), (0, 0), (0, 0)))
    sup = jnp.transpose(sup3, (1, 0, 2)).reshape(KSHOT * NP, D_OUT)
    qf = f3[:, KSHOT:].reshape(NQ, D_OUT)
    qf = jnp.pad(qf, ((0, NQP - NQ), (0, 0)))

    qn, apn = pl.pallas_call(
        _proto_kernel,
        out_shape=[
            jax.ShapeDtypeStruct((NQP, D_OUT), f32),
            jax.ShapeDtypeStruct((NP, D_OUT), f32),
        ],
    )(sup, qf)

    out = pl.pallas_call(
        _knn_kernel,
        grid=(2 * NB,),
        in_specs=[
            pl.BlockSpec((BLK, D_OUT), lambda i: (jnp.where(i < NB, i, NB - 1), 0)),
            pl.BlockSpec((NQP, D_OUT), lambda i: (0, 0)),
            pl.BlockSpec((NQP, D_OUT), lambda i: (0, 0)),
            pl.BlockSpec((NP, D_OUT), lambda i: (0, 0)),
            pl.BlockSpec((1, 1), lambda i: (0, 0)),
        ],
        out_specs=pl.BlockSpec((BLK, NP), lambda i: (jnp.where(i < NB, 0, i - NB), 0)),
        out_shape=jax.ShapeDtypeStruct((NQP, NP), f32),
        scratch_shapes=[
            pltpu.VMEM((NQP, NQP), f32),
            pltpu.VMEM((NQP, 1), f32),
            pltpu.VMEM((1, NQP), f32),
        ],
    )(qn, qn, qf, apn, tao.reshape(1, 1))

    return out[:NQ, :N]
